# baseline (speedup 1.0000x reference)
import jax
import jax.numpy as jnp
from jax import lax
from jax.experimental import pallas as pl
from jax.experimental.pallas import tpu as pltpu
from jax.experimental.pallas import tpu_sc as plsc

V, D, B, L = 100000, 128, 4096, 50
NC, NS, LANES = 2, 16, 16
NW = NC * NS
BPW = B // NW
C = 8
ROWS = C * L
NCH = BPW // C
KD = D // LANES
PAD = 17
GROUPS = (0, 16, 32, 34)


def _body(center_hbm, ctx_hbm, in_hbm, out_hbm, score_hbm,
          cidx_all, ctx_idx_all, vrows0, vrows1, urows0, urows1,
          pad, score0, score1, sem0, sem1):
    wid = lax.axis_index("s") * NC + lax.axis_index("c")
    iota = lax.iota(jnp.int32, LANES)

    pltpu.sync_copy(center_hbm.at[pl.ds(wid * BPW, BPW)], cidx_all)
    pltpu.sync_copy(ctx_hbm.at[pl.ds(wid * BPW * L, BPW * L)], ctx_idx_all)

    def issue(ch, vr, ur, sem):
        pltpu.async_copy(in_hbm.at[cidx_all.at[pl.ds(ch * C, C)]], vr, sem)
        pltpu.async_copy(out_hbm.at[ctx_idx_all.at[pl.ds(ch * ROWS, ROWS)]],
                         ur, sem)

    def wait(vr, ur, sem):
        pltpu.make_async_copy(in_hbm.at[pl.ds(0, C)], vr, sem).wait()
        pltpu.make_async_copy(out_hbm.at[pl.ds(0, ROWS)], ur, sem).wait()

    bufs = ((vrows0, urows0, score0, sem0), (vrows1, urows1, score1, sem1))
    issue(0, vrows0, urows0, sem0)

    @pl.loop(0, NCH, step=2)
    def _outer(ch0):
        for j in range(2):
            ch = ch0 + j
            vr, ur, sc, sem = bufs[j]
            nvr, nur, _, nsem = bufs[1 - j]

            @pl.when(ch + 1 < NCH)
            def _prefetch():
                issue(ch + 1, nvr, nur, nsem)

            wait(vr, ur, sem)

            @pl.loop(0, C)
            def _b(b):
                vvecs = [vr[b, pl.ds(k * LANES, LANES)] for k in range(KD)]
                results = []
                for gi, s in enumerate(GROUPS):
                    pb = (gi % 2) * (LANES * PAD)
                    parts = []
                    for r in range(LANES):
                        row = b * L + s + r
                        prods = [vvecs[k] * ur[row, pl.ds(k * LANES, LANES)]
                                 for k in range(KD)]
                        while len(prods) > 1:
                            prods = [prods[i] + prods[i + 1]
                                     for i in range(0, len(prods), 2)]
                        parts.append(prods[0])
                    for r in range(LANES):
                        pad[pl.ds(pb + r * PAD, LANES)] = parts[r]
                    accs = [plsc.load_gather(pad, [pb + iota * PAD + d2])
                            for d2 in range(LANES)]
                    while len(accs) > 1:
                        accs = [accs[i] + accs[i + 1]
                                for i in range(0, len(accs), 2)]
                    results.append(accs[0])
                for gi, s in enumerate(GROUPS):
                    sc[pl.ds(b * L + s, LANES)] = results[gi]

            pltpu.sync_copy(sc, score_hbm.at[pl.ds((wid * BPW + ch * C) * L,
                                                   ROWS)])


def kernel(center, context, in_em, out_em):
    ctx_flat = context.reshape(B * L).astype(jnp.int32)
    center32 = center.astype(jnp.int32)
    mesh = plsc.VectorSubcoreMesh(core_axis_name="c", subcore_axis_name="s")
    score = pl.kernel(
        _body,
        out_type=jax.ShapeDtypeStruct((B * L,), jnp.float32),
        mesh=mesh,
        compiler_params=pltpu.CompilerParams(needs_layout_passes=False),
        scratch_types=[
            pltpu.VMEM((BPW,), jnp.int32),
            pltpu.VMEM((BPW * L,), jnp.int32),
            pltpu.VMEM((C, D), jnp.float32),
            pltpu.VMEM((C, D), jnp.float32),
            pltpu.VMEM((ROWS, D), jnp.float32),
            pltpu.VMEM((ROWS, D), jnp.float32),
            pltpu.VMEM((2 * LANES * PAD,), jnp.float32),
            pltpu.VMEM((ROWS,), jnp.float32),
            pltpu.VMEM((ROWS,), jnp.float32),
            pltpu.SemaphoreType.DMA,
            pltpu.SemaphoreType.DMA,
        ],
    )(center32, ctx_flat, in_em, out_em)
    return score.reshape(B, L)


# speedup vs baseline: 1.0755x; 1.0755x over previous
import jax
import jax.numpy as jnp
from jax import lax
from jax.experimental import pallas as pl
from jax.experimental.pallas import tpu as pltpu
from jax.experimental.pallas import tpu_sc as plsc

V, D, B, L = 100000, 128, 4096, 50
NC, NS, LANES = 2, 16, 16
NW = NC * NS
BPW = B // NW
C = 8
ROWS = C * L
NCH = BPW // C
KD = D // LANES
PAD = 17
GROUPS = (0, 16, 32, 34)


def _body(center_hbm, ctx_hbm, in_hbm, out_hbm, score_hbm,
          cidx_all, ctx_idx_all, vrows0, vrows1, urows0, urows1,
          pad, score0, score1, sem0, sem1):
    wid = lax.axis_index("s") * NC + lax.axis_index("c")
    iota = lax.iota(jnp.int32, LANES)

    pltpu.sync_copy(center_hbm.at[pl.ds(wid * BPW, BPW)], cidx_all)
    pltpu.sync_copy(ctx_hbm.at[pl.ds(wid * BPW * L, BPW * L)], ctx_idx_all)

    def issue(ch, vr, ur, sem):
        pltpu.async_copy(in_hbm.at[cidx_all.at[pl.ds(ch * C, C)]], vr, sem)
        pltpu.async_copy(out_hbm.at[ctx_idx_all.at[pl.ds(ch * ROWS, ROWS)]],
                         ur, sem)

    def wait(vr, ur, sem):
        pltpu.make_async_copy(in_hbm.at[pl.ds(0, C)], vr, sem).wait()
        pltpu.make_async_copy(out_hbm.at[pl.ds(0, ROWS)], ur, sem).wait()

    bufs = ((vrows0, urows0, score0, sem0), (vrows1, urows1, score1, sem1))
    issue(0, vrows0, urows0, sem0)

    @pl.loop(0, NCH, step=2)
    def _outer(ch0):
        for j in range(2):
            ch = ch0 + j
            vr, ur, sc, sem = bufs[j]
            nvr, nur, _, nsem = bufs[1 - j]

            @pl.when(ch + 1 < NCH)
            def _prefetch():
                issue(ch + 1, nvr, nur, nsem)

            wait(vr, ur, sem)

            @pl.loop(0, C)
            def _b(b):
                vvecs = [vr[b, pl.ds(k * LANES, LANES)] for k in range(KD)]
                for s in GROUPS:
                    score_vec = jnp.zeros((LANES,), jnp.float32)
                    for r in range(LANES):
                        row = b * L + s + r
                        prods = [vvecs[k] * ur[row, pl.ds(k * LANES, LANES)]
                                 for k in range(KD)]
                        while len(prods) > 1:
                            prods = [prods[i] + prods[i + 1]
                                     for i in range(0, len(prods), 2)]
                        score_vec = jnp.where(iota == r, jnp.sum(prods[0]),
                                              score_vec)
                    sc[pl.ds(b * L + s, LANES)] = score_vec

            pltpu.sync_copy(sc, score_hbm.at[pl.ds((wid * BPW + ch * C) * L,
                                                   ROWS)])


def kernel(center, context, in_em, out_em):
    ctx_flat = context.reshape(B * L).astype(jnp.int32)
    center32 = center.astype(jnp.int32)
    mesh = plsc.VectorSubcoreMesh(core_axis_name="c", subcore_axis_name="s")
    score = pl.kernel(
        _body,
        out_type=jax.ShapeDtypeStruct((B * L,), jnp.float32),
        mesh=mesh,
        compiler_params=pltpu.CompilerParams(needs_layout_passes=False),
        scratch_types=[
            pltpu.VMEM((BPW,), jnp.int32),
            pltpu.VMEM((BPW * L,), jnp.int32),
            pltpu.VMEM((C, D), jnp.float32),
            pltpu.VMEM((C, D), jnp.float32),
            pltpu.VMEM((ROWS, D), jnp.float32),
            pltpu.VMEM((ROWS, D), jnp.float32),
            pltpu.VMEM((LANES * PAD,), jnp.float32),
            pltpu.VMEM((ROWS,), jnp.float32),
            pltpu.VMEM((ROWS,), jnp.float32),
            pltpu.SemaphoreType.DMA,
            pltpu.SemaphoreType.DMA,
        ],
    )(center32, ctx_flat, in_em, out_em)
    return score.reshape(B, L)
